# Initial kernel scaffold; baseline (speedup 1.0000x reference)
import functools

import jax
import jax.numpy as jnp
from jax import lax
from jax.experimental import pallas as pl
from jax.experimental.pallas import tpu as pltpu
from jax.experimental.pallas import tpu_sc as plsc

DIM = 64
CHUNK = 128
NBUF = 4
NC, NS = 2, 16
NW = NC * NS


@functools.lru_cache(maxsize=None)
def _make_gather(n_idx_rows, n_table_rows):
    rows_per_w = n_idx_rows // NW
    per_w = rows_per_w * CHUNK
    assert rows_per_w * NW == n_idx_rows
    assert rows_per_w % NBUF == 0

    mesh = plsc.VectorSubcoreMesh(core_axis_name="c", subcore_axis_name="s")

    @functools.partial(
        pl.kernel,
        out_type=jax.ShapeDtypeStruct((n_idx_rows * CHUNK, DIM), jnp.float32),
        mesh=mesh,
        scratch_types=[
            pltpu.VMEM((rows_per_w, CHUNK), jnp.int32),
            pltpu.VMEM((NBUF, CHUNK, DIM), jnp.float32),
        ]
        + [pltpu.SemaphoreType.DMA] * NBUF
        + [pltpu.SemaphoreType.DMA] * NBUF,
    )
    def gather_kernel(table_hbm, idx_hbm, out_hbm, idx_v, rows_v, *sems):
        gsems = sems[:NBUF]
        ssems = sems[NBUF:]
        wid = lax.axis_index("s") * NC + lax.axis_index("c")
        base = wid * per_w

        pltpu.sync_copy(idx_hbm.at[pl.ds(wid * rows_per_w, rows_per_w)], idx_v)

        def gather(j, b):
            return pltpu.async_copy(
                table_hbm.at[idx_v.at[j]], rows_v.at[b], gsems[b])

        def wait_gather(b):
            pltpu.make_async_copy(
                table_hbm.at[pl.ds(0, CHUNK)], rows_v.at[b], gsems[b]).wait()

        def store(j, b):
            return pltpu.async_copy(
                rows_v.at[b],
                out_hbm.at[pl.ds(base + j * CHUNK, CHUNK)],
                ssems[b])

        for b in range(NBUF):
            gather(b, b)

        @pl.loop(0, rows_per_w, step=NBUF)
        def _(j0):
            handles = []
            for b in range(NBUF):
                wait_gather(b)
                handles.append(store(j0 + b, b))
            for b in range(NBUF):
                handles[b].wait()
                nxt = j0 + b + NBUF

                @pl.when(nxt < rows_per_w)
                def _():
                    gather(nxt, b)

    return gather_kernel


def kernel(node_positions, psne_layer):
    b, s = node_positions.shape
    n = b * s
    idx2d = node_positions.reshape(n // CHUNK, CHUNK).astype(jnp.int32)
    fn = _make_gather(n // CHUNK, psne_layer.shape[0])
    out = fn(psne_layer, idx2d)
    return out.reshape(b, s, DIM)


# speedup vs baseline: 6.1966x; 6.1966x over previous
import functools

import jax
import jax.numpy as jnp
from jax import lax
from jax.experimental import pallas as pl
from jax.experimental.pallas import tpu as pltpu
from jax.experimental.pallas import tpu_sc as plsc

DIM = 64
CHUNK = 128
NBUF = 4
NC, NS = 2, 16
NW = NC * NS


@functools.lru_cache(maxsize=None)
def _make_gather(n_idx_rows, n_table_rows):
    rows_per_w = n_idx_rows // NW
    per_w = rows_per_w * CHUNK
    assert rows_per_w * NW == n_idx_rows
    assert rows_per_w % NBUF == 0

    mesh = plsc.VectorSubcoreMesh(core_axis_name="c", subcore_axis_name="s")

    @functools.partial(
        pl.kernel,
        out_type=jax.ShapeDtypeStruct((n_idx_rows * CHUNK, DIM), jnp.float32),
        mesh=mesh,
        scratch_types=[
            pltpu.VMEM((rows_per_w, CHUNK), jnp.int32),
            pltpu.VMEM((NBUF, CHUNK, DIM), jnp.float32),
        ]
        + [pltpu.SemaphoreType.DMA] * NBUF
        + [pltpu.SemaphoreType.DMA] * NBUF,
        compiler_params=pltpu.CompilerParams(use_tc_tiling_on_sc=False),
    )
    def gather_kernel(table_hbm, idx_hbm, out_hbm, idx_v, rows_v, *sems):
        gsems = sems[:NBUF]
        ssems = sems[NBUF:]
        wid = lax.axis_index("s") * NC + lax.axis_index("c")
        base = wid * per_w

        pltpu.sync_copy(idx_hbm.at[pl.ds(wid * rows_per_w, rows_per_w)], idx_v)

        def gather(j, b):
            return pltpu.async_copy(
                table_hbm.at[idx_v.at[j]], rows_v.at[b], gsems[b])

        def wait_gather(b):
            pltpu.make_async_copy(
                table_hbm.at[pl.ds(0, CHUNK)], rows_v.at[b], gsems[b]).wait()

        def store(j, b):
            return pltpu.async_copy(
                rows_v.at[b],
                out_hbm.at[pl.ds(base + j * CHUNK, CHUNK)],
                ssems[b])

        for b in range(NBUF):
            gather(b, b)

        @pl.loop(0, rows_per_w, step=NBUF)
        def _(j0):
            handles = []
            for b in range(NBUF):
                wait_gather(b)
                handles.append(store(j0 + b, b))
            for b in range(NBUF):
                handles[b].wait()
                nxt = j0 + b + NBUF

                @pl.when(nxt < rows_per_w)
                def _():
                    gather(nxt, b)

    return gather_kernel


def kernel(node_positions, psne_layer):
    b, s = node_positions.shape
    n = b * s
    idx2d = node_positions.reshape(n // CHUNK, CHUNK).astype(jnp.int32)
    fn = _make_gather(n // CHUNK, psne_layer.shape[0])
    out = fn(psne_layer, idx2d)
    return out.reshape(b, s, DIM)


# speedup vs baseline: 6.2007x; 1.0007x over previous
import functools

import jax
import jax.numpy as jnp
from jax import lax
from jax.experimental import pallas as pl
from jax.experimental.pallas import tpu as pltpu
from jax.experimental.pallas import tpu_sc as plsc

DIM = 64
CHUNK = 256
NBUF = 4
NC, NS = 2, 16
NW = NC * NS


@functools.lru_cache(maxsize=None)
def _make_gather(n_idx_rows, n_table_rows):
    rows_per_w = n_idx_rows // NW
    per_w = rows_per_w * CHUNK
    assert rows_per_w * NW == n_idx_rows
    assert rows_per_w % NBUF == 0

    mesh = plsc.VectorSubcoreMesh(core_axis_name="c", subcore_axis_name="s")

    @functools.partial(
        pl.kernel,
        out_type=jax.ShapeDtypeStruct((n_idx_rows * CHUNK, DIM), jnp.float32),
        mesh=mesh,
        scratch_types=[
            pltpu.VMEM((rows_per_w, CHUNK), jnp.int32),
            pltpu.VMEM((NBUF, CHUNK, DIM), jnp.float32),
        ]
        + [pltpu.SemaphoreType.DMA] * NBUF
        + [pltpu.SemaphoreType.DMA] * NBUF,
        compiler_params=pltpu.CompilerParams(use_tc_tiling_on_sc=False),
    )
    def gather_kernel(table_hbm, idx_hbm, out_hbm, idx_v, rows_v, *sems):
        gsems = sems[:NBUF]
        ssems = sems[NBUF:]
        wid = lax.axis_index("s") * NC + lax.axis_index("c")
        base = wid * per_w

        pltpu.sync_copy(idx_hbm.at[pl.ds(wid * rows_per_w, rows_per_w)], idx_v)

        def gather(j, b):
            return pltpu.async_copy(
                table_hbm.at[idx_v.at[j]], rows_v.at[b], gsems[b])

        def wait_gather(b):
            pltpu.make_async_copy(
                table_hbm.at[pl.ds(0, CHUNK)], rows_v.at[b], gsems[b]).wait()

        def store(j, b):
            return pltpu.async_copy(
                rows_v.at[b],
                out_hbm.at[pl.ds(base + j * CHUNK, CHUNK)],
                ssems[b])

        for b in range(NBUF):
            gather(b, b)

        @pl.loop(0, rows_per_w, step=NBUF)
        def _(j0):
            handles = []
            for b in range(NBUF):
                wait_gather(b)
                handles.append(store(j0 + b, b))
            for b in range(NBUF):
                handles[b].wait()
                nxt = j0 + b + NBUF

                @pl.when(nxt < rows_per_w)
                def _():
                    gather(nxt, b)

    return gather_kernel


def kernel(node_positions, psne_layer):
    b, s = node_positions.shape
    n = b * s
    idx2d = node_positions.reshape(n // CHUNK, CHUNK).astype(jnp.int32)
    fn = _make_gather(n // CHUNK, psne_layer.shape[0])
    out = fn(psne_layer, idx2d)
    return out.reshape(b, s, DIM)


# speedup vs baseline: 6.6118x; 1.0663x over previous
import functools

import jax
import jax.numpy as jnp
from jax import lax
from jax.experimental import pallas as pl
from jax.experimental.pallas import tpu as pltpu
from jax.experimental.pallas import tpu_sc as plsc

DIM = 64
PDIM = 128
SEQ = 50
GB = 4
HB = GB // 2
NC, NS = 2, 16
NW = NC * NS


@functools.lru_cache(maxsize=None)
def _make_gather(n_batch, n_table_rows):
    b_per_w = n_batch // NW
    n_groups = b_per_w // GB
    idx_per_w = b_per_w * SEQ
    grows = GB * SEQ
    assert b_per_w * NW == n_batch
    assert n_groups * GB == b_per_w
    assert n_groups % 2 == 0 and grows % 8 == 0

    mesh = plsc.VectorSubcoreMesh(core_axis_name="c", subcore_axis_name="s")

    @functools.partial(
        pl.kernel,
        out_type=jax.ShapeDtypeStruct((n_batch, SEQ, DIM), jnp.float32),
        mesh=mesh,
        scratch_types=[
            pltpu.VMEM((idx_per_w,), jnp.int32),
            pltpu.VMEM((2, grows, PDIM), jnp.float32),
            pltpu.VMEM((2, HB, SEQ, DIM), jnp.float32),
            pltpu.SemaphoreType.DMA,
            pltpu.SemaphoreType.DMA,
            pltpu.SemaphoreType.DMA,
            pltpu.SemaphoreType.DMA,
        ],
    )
    def gather_kernel(table_hbm, idx_hbm, out_hbm, idx_v, rows_v, pack_v,
                      gsem0, gsem1, psem0, psem1):
        gsems = (gsem0, gsem1)
        psems = (psem0, psem1)
        wid = lax.axis_index("s") * NC + lax.axis_index("c")
        b0 = wid * b_per_w

        pltpu.sync_copy(idx_hbm.at[pl.ds(wid * idx_per_w, idx_per_w)], idx_v)

        def gather(g, buf):
            pltpu.async_copy(
                table_hbm.at[idx_v.at[pl.ds(g * grows, grows)]],
                rows_v.at[buf], gsems[buf])

        def wait_gather(buf):
            pltpu.make_async_copy(
                table_hbm.at[pl.ds(0, grows)], rows_v.at[buf],
                gsems[buf]).wait()

        def repack(buf, h):
            for i in range(HB):
                base = (h * HB + i) * SEQ

                @pl.loop(0, SEQ, unroll=2)
                def _(s):
                    for c in range(DIM // 16):
                        pack_v[h, i, s, pl.ds(c * 16, 16)] = (
                            rows_v[buf, base + s, pl.ds(c * 16, 16)])

        def store(g, h):
            pltpu.async_copy(
                pack_v.at[h],
                out_hbm.at[pl.ds(b0 + g * GB + h * HB, HB)],
                psems[h])

        def wait_store(h):
            pltpu.make_async_copy(
                pack_v.at[h], out_hbm.at[pl.ds(b0, HB)], psems[h]).wait()

        gather(0, 0)
        gather(1, 1)

        @pl.loop(0, n_groups, step=2)
        def _(j0):
            for buf in range(2):
                j = j0 + buf
                wait_gather(buf)
                for h in range(2):

                    @pl.when(j > 0)
                    def _():
                        wait_store(h)

                    repack(buf, h)
                    store(j, h)

                @pl.when(j < n_groups - 2)
                def _():
                    gather(j + 2, buf)

        wait_store(0)
        wait_store(1)

    return gather_kernel


def kernel(node_positions, psne_layer):
    b, s = node_positions.shape
    idx_flat = node_positions.reshape(b * s).astype(jnp.int32)
    table128 = jnp.pad(psne_layer, ((0, 0), (0, PDIM - DIM)))
    fn = _make_gather(b, psne_layer.shape[0])
    return fn(table128, idx_flat)
